# baseline (speedup 1.0000x reference)
import functools

import jax
import jax.numpy as jnp
from jax import lax
from jax.experimental import pallas as pl
from jax.experimental.pallas import tpu as pltpu
from jax.experimental.pallas import tpu_sc as plsc

F32 = jnp.float32
_HIGH = lax.Precision.HIGHEST

_NC = 2
_NS = 16
_NW = _NC * _NS
_CH = 128
_DEGW = 16

_TC_PARAMS = pltpu.CompilerParams(dimension_semantics=("parallel",))
_SC_PARAMS = pltpu.CompilerParams(use_tc_tiling_on_sc=False)


def _dot(a, b):
    return jnp.dot(a, b, preferred_element_type=F32, precision=_HIGH)


def _mlp3_body(x_ref, w1, b1, w2, b2, w3, b3, o_ref):
    h = jnp.maximum(_dot(x_ref[...], w1[...]) + b1[...], 0.0)
    h = jnp.maximum(_dot(h, w2[...]) + b2[...], 0.0)
    o_ref[...] = _dot(h, w3[...]) + b3[...]


def _dinv_g1_body(degp_ref, emb_ref, wg1, dinv_ref, g1_ref):
    deg = degp_ref[0] + degp_ref[1] + 1.0
    dinv = lax.rsqrt(deg)
    dinv_ref[...] = dinv
    g1_ref[...] = dinv[:, :1] * _dot(emb_ref[...], wg1[...])


def _post_body(p_ref, g_ref, dinv_ref, bg, wgn, gn_ref):
    dinv = dinv_ref[...][:, :1]
    h = jnp.maximum(dinv * (p_ref[0] + p_ref[1] + g_ref[...]) + bg[...], 0.0)
    gn_ref[...] = dinv * _dot(h, wgn[...])


def _dec_body(p_ref, g_ref, dinv_ref, bg, emb_ref, wf, bf, wd1, bd1, wd2, bd2,
              o_ref):
    dinv = dinv_ref[...][:, :1]
    h3 = dinv * (p_ref[0] + p_ref[1] + g_ref[...]) + bg[...]
    cat = jnp.concatenate([emb_ref[...], h3], axis=-1)
    o = _dot(jnp.maximum(cat, 0.0), wf[...]) + bf[...]
    d = jnp.maximum(_dot(o, wd1[...]) + bd1[...], 0.0)
    logits = _dot(d, wd2[...]) + bd2[...]
    m = jnp.max(logits, axis=-1, keepdims=True)
    e = jnp.exp(logits - m)
    o_ref[...] = e / jnp.sum(e, axis=-1, keepdims=True)


def _full(shape):
    return pl.BlockSpec(shape, lambda i: (0,) * len(shape))


def _rows(rb, *rest):
    n = len(rest)
    return pl.BlockSpec((rb,) + rest, lambda i: (i,) + (0,) * n)


def _rows3(lead, rb, *rest):
    n = len(rest)
    return pl.BlockSpec((lead, rb) + rest, lambda i: (0, i) + (0,) * n)


_NBUF = 3


def _edge_body(cpw, rps, g_hbm, src_hbm, dst_hbm, zero_hbm, out_hbm,
               src_v, dst_v, *rest):
    bufs = rest[:_NBUF]
    g_spm, acc = rest[_NBUF], rest[_NBUF + 1]
    gsems = rest[_NBUF + 2:2 * _NBUF + 2]
    ssems = rest[2 * _NBUF + 2:]
    cid = lax.axis_index("c")
    sid = lax.axis_index("s")
    wid = sid * _NC + cid
    pltpu.sync_copy(src_hbm.at[pl.ds(wid * cpw, cpw)], src_v)
    pltpu.sync_copy(dst_hbm.at[pl.ds(wid * cpw, cpw)], dst_v)
    pltpu.sync_copy(g_hbm.at[pl.ds(sid * rps, rps)],
                    g_spm.at[pl.ds(sid * rps, rps)])
    pltpu.sync_copy(zero_hbm.at[pl.ds(sid * rps, rps)],
                    acc.at[pl.ds(sid * rps, rps)])
    plsc.subcore_barrier()

    def gather(j, b, sem):
        return pltpu.make_async_copy(g_spm.at[src_v.at[j]], bufs[b], sem)

    def scatter(j, b, sem):
        return pltpu.make_async_copy(bufs[b], acc.at[dst_v.at[j]], sem)

    for b in range(_NBUF):
        gather(b, b, gsems[b]).start()

    @pl.loop(0, cpw, step=_NBUF)
    def _(j):
        for b in range(_NBUF):
            gather(j + b, b, gsems[b]).wait()
            pltpu.async_copy(bufs[b], acc.at[dst_v.at[j + b]], ssems[b],
                             add=True)
        for b in range(_NBUF):
            @pl.when(j + _NBUF + b < cpw)
            def _(b=b):
                scatter(j + b, b, ssems[b]).wait()
                gather(j + _NBUF + b, b, gsems[b]).start()

    for b in range(_NBUF):
        scatter(cpw - _NBUF + b, b, ssems[b]).wait()

    plsc.subcore_barrier()
    pltpu.sync_copy(acc.at[pl.ds(sid * rps, rps)],
                    out_hbm.at[cid].at[pl.ds(sid * rps, rps)])


def _deg_body(cpw, rps, dst_hbm, zero_hbm, ones_hbm, out_hbm,
              dst_v, ones_v, acc):
    cid = lax.axis_index("c")
    sid = lax.axis_index("s")
    wid = sid * _NC + cid
    pltpu.sync_copy(dst_hbm.at[pl.ds(wid * cpw, cpw)], dst_v)
    pltpu.sync_copy(ones_hbm, ones_v)
    pltpu.sync_copy(zero_hbm.at[pl.ds(sid * rps, rps)],
                    acc.at[pl.ds(sid * rps, rps)])
    plsc.subcore_barrier()

    @pl.loop(0, cpw)
    def _(j):
        pltpu.sync_copy(ones_v, acc.at[dst_v.at[j]], add=True)

    plsc.subcore_barrier()
    pltpu.sync_copy(acc.at[pl.ds(sid * rps, rps)],
                    out_hbm.at[cid].at[pl.ds(sid * rps, rps)])


def kernel(x, W1, b1, W2, b2, W3, b3, Wg1, bg1, Wg2, bg2, Wg3, bg3,
           Wf, bf, Wd1, bd1, Wd2, bd2, edge_index):
    N, din = x.shape
    L = Wg1.shape[0]
    C = Wd2.shape[1]
    E = edge_index.shape[1]

    RB = 1280
    NP = -(-N // RB) * RB
    if NP - N < _DEGW:
        NP += RB
    GRID = NP // RB
    RPS = NP // _NS

    cpw = -(-E // (_NW * _CH))
    cpw = -(-cpw // _NBUF) * _NBUF
    EP = _NW * cpw * _CH
    NCH = EP // _CH

    src = edge_index[0]
    dst = edge_index[1]
    srcp = jnp.concatenate(
        [src, jnp.zeros((EP - E,), src.dtype)]).reshape(NCH, _CH)
    dstp = jnp.concatenate(
        [dst, jnp.full((EP - E,), N, dst.dtype)]).reshape(NCH, _CH)

    xp = jnp.pad(x, ((0, NP - N), (0, 0)))
    zeros_l = jnp.zeros((NP, L), F32)
    zeros_d = jnp.zeros((NP, _DEGW), F32)
    ones_d = jnp.ones((_CH, _DEGW), F32)

    b1r, b2r, b3r = b1[None, :], b2[None, :], b3[None, :]
    bg1r, bg2r, bg3r = bg1[None, :], bg2[None, :], bg3[None, :]
    bfr, bd1r, bd2r = bf[None, :], bd1[None, :], bd2[None, :]

    init_embed = pl.pallas_call(
        _mlp3_body,
        grid=(GRID,),
        in_specs=[_rows(RB, din), _full(W1.shape), _full((1, 512)),
                  _full(W2.shape), _full((1, 256)),
                  _full(W3.shape), _full((1, L))],
        out_specs=_rows(RB, L),
        out_shape=jax.ShapeDtypeStruct((NP, L), F32),
        compiler_params=_TC_PARAMS,
    )(xp, W1, b1r, W2, b2r, W3, b3r)

    mesh = plsc.VectorSubcoreMesh(core_axis_name="c", subcore_axis_name="s")

    degp = pl.kernel(
        functools.partial(_deg_body, cpw, RPS),
        out_type=jax.ShapeDtypeStruct((_NC, NP, _DEGW), F32),
        mesh=mesh,
        scratch_types=[
            pltpu.VMEM((cpw, _CH), jnp.int32),
            pltpu.VMEM((_CH, _DEGW), F32),
            pltpu.VMEM_SHARED((NP, _DEGW), F32),
        ],
        compiler_params=_SC_PARAMS,
    )(dstp, zeros_d, ones_d)

    dinv16, g = pl.pallas_call(
        _dinv_g1_body,
        grid=(GRID,),
        in_specs=[_rows3(_NC, RB, _DEGW), _rows(RB, L), _full(Wg1.shape)],
        out_specs=[_rows(RB, _DEGW), _rows(RB, L)],
        out_shape=[jax.ShapeDtypeStruct((NP, _DEGW), F32),
                   jax.ShapeDtypeStruct((NP, L), F32)],
        compiler_params=_TC_PARAMS,
    )(degp, init_embed, Wg1)

    edge_call = pl.kernel(
        functools.partial(_edge_body, cpw, RPS),
        out_type=jax.ShapeDtypeStruct((_NC, NP, L), F32),
        mesh=mesh,
        scratch_types=(
            [pltpu.VMEM((cpw, _CH), jnp.int32),
             pltpu.VMEM((cpw, _CH), jnp.int32)]
            + [pltpu.VMEM((_CH, L), F32)] * _NBUF
            + [pltpu.VMEM_SHARED((NP, L), F32),
               pltpu.VMEM_SHARED((NP, L), F32)]
            + [pltpu.SemaphoreType.DMA] * (2 * _NBUF)
        ),
        compiler_params=_SC_PARAMS,
    )

    def post_call(p, g_cur, bgr, wgn):
        return pl.pallas_call(
            _post_body,
            grid=(GRID,),
            in_specs=[_rows3(_NC, RB, L), _rows(RB, L), _rows(RB, _DEGW),
                      _full((1, L)), _full(wgn.shape)],
            out_specs=_rows(RB, L),
            out_shape=jax.ShapeDtypeStruct((NP, L), F32),
            compiler_params=_TC_PARAMS,
        )(p, g_cur, dinv16, bgr, wgn)

    p = edge_call(g, srcp, dstp, zeros_l)
    g = post_call(p, g, bg1r, Wg2)
    p = edge_call(g, srcp, dstp, zeros_l)
    g = post_call(p, g, bg2r, Wg3)
    p = edge_call(g, srcp, dstp, zeros_l)

    out = pl.pallas_call(
        _dec_body,
        grid=(GRID,),
        in_specs=[_rows3(_NC, RB, L), _rows(RB, L), _rows(RB, _DEGW),
                  _full((1, L)), _rows(RB, L), _full(Wf.shape), _full((1, L)),
                  _full(Wd1.shape), _full((1, L)), _full(Wd2.shape),
                  _full((1, C))],
        out_specs=_rows(RB, C),
        out_shape=jax.ShapeDtypeStruct((NP, C), F32),
        compiler_params=_TC_PARAMS,
    )(p, g, dinv16, bg3r, init_embed, Wf, bfr, Wd1, bd1r, Wd2, bd2r)

    return out[:N]


# speedup vs baseline: 1.3264x; 1.3264x over previous
import functools

import jax
import jax.numpy as jnp
from jax import lax
from jax.experimental import pallas as pl
from jax.experimental.pallas import tpu as pltpu
from jax.experimental.pallas import tpu_sc as plsc

F32 = jnp.float32
_HIGH = lax.Precision.DEFAULT

_NC = 2
_NS = 16
_NW = _NC * _NS
_CH = 128
_DEGW = 16

_TC_PARAMS = pltpu.CompilerParams(dimension_semantics=("parallel",))
_SC_PARAMS = pltpu.CompilerParams(use_tc_tiling_on_sc=False)


def _dot(a, b):
    return jnp.dot(a, b, preferred_element_type=F32, precision=_HIGH)


def _mlp3_body(x_ref, w1, b1, w2, b2, w3, b3, o_ref):
    h = jnp.maximum(_dot(x_ref[...], w1[...]) + b1[...], 0.0)
    h = jnp.maximum(_dot(h, w2[...]) + b2[...], 0.0)
    o_ref[...] = _dot(h, w3[...]) + b3[...]


def _dinv_g1_body(degp_ref, emb_ref, wg1, dinv_ref, g1_ref):
    deg = degp_ref[0] + degp_ref[1] + 1.0
    dinv = lax.rsqrt(deg)
    dinv_ref[...] = dinv
    g1_ref[...] = dinv[:, :1] * _dot(emb_ref[...], wg1[...])


def _post_body(p_ref, g_ref, dinv_ref, bg, wgn, gn_ref):
    dinv = dinv_ref[...][:, :1]
    h = jnp.maximum(dinv * (p_ref[0] + p_ref[1] + g_ref[...]) + bg[...], 0.0)
    gn_ref[...] = dinv * _dot(h, wgn[...])


def _dec_body(p_ref, g_ref, dinv_ref, bg, emb_ref, wf, bf, wd1, bd1, wd2, bd2,
              o_ref):
    dinv = dinv_ref[...][:, :1]
    h3 = dinv * (p_ref[0] + p_ref[1] + g_ref[...]) + bg[...]
    cat = jnp.concatenate([emb_ref[...], h3], axis=-1)
    o = _dot(jnp.maximum(cat, 0.0), wf[...]) + bf[...]
    d = jnp.maximum(_dot(o, wd1[...]) + bd1[...], 0.0)
    logits = _dot(d, wd2[...]) + bd2[...]
    m = jnp.max(logits, axis=-1, keepdims=True)
    e = jnp.exp(logits - m)
    o_ref[...] = e / jnp.sum(e, axis=-1, keepdims=True)


def _full(shape):
    return pl.BlockSpec(shape, lambda i: (0,) * len(shape))


def _rows(rb, *rest):
    n = len(rest)
    return pl.BlockSpec((rb,) + rest, lambda i: (i,) + (0,) * n)


def _rows3(lead, rb, *rest):
    n = len(rest)
    return pl.BlockSpec((lead, rb) + rest, lambda i: (0, i) + (0,) * n)


_NBUF = 2


def _edge_body(cpw, rps, g_hbm, src_hbm, dst_hbm, zero_hbm, out_hbm,
               src_v, dst_v, *rest):
    bufs = rest[:_NBUF]
    g_spm, acc = rest[_NBUF], rest[_NBUF + 1]
    gsems = rest[_NBUF + 2:2 * _NBUF + 2]
    ssems = rest[2 * _NBUF + 2:]
    cid = lax.axis_index("c")
    sid = lax.axis_index("s")
    wid = sid * _NC + cid
    pltpu.sync_copy(src_hbm.at[pl.ds(wid * cpw, cpw)], src_v)
    pltpu.sync_copy(dst_hbm.at[pl.ds(wid * cpw, cpw)], dst_v)
    pltpu.sync_copy(g_hbm.at[pl.ds(sid * rps, rps)],
                    g_spm.at[pl.ds(sid * rps, rps)])
    pltpu.sync_copy(zero_hbm.at[pl.ds(sid * rps, rps)],
                    acc.at[pl.ds(sid * rps, rps)])
    plsc.subcore_barrier()

    buf_a, buf_b = bufs[0], bufs[1]
    sem_a, sem_b = gsems[0], gsems[1]
    pltpu.make_async_copy(g_spm.at[src_v.at[0]], buf_a, sem_a).start()

    @pl.loop(0, cpw, step=2)
    def _(j):
        pltpu.make_async_copy(g_spm.at[src_v.at[j + 1]], buf_b, sem_b).start()
        pltpu.make_async_copy(g_spm.at[src_v.at[j]], buf_a, sem_a).wait()
        pltpu.sync_copy(buf_a, acc.at[dst_v.at[j]], add=True)

        @pl.when(j + 2 < cpw)
        def _():
            pltpu.make_async_copy(g_spm.at[src_v.at[j + 2]], buf_a,
                                  sem_a).start()

        pltpu.make_async_copy(g_spm.at[src_v.at[j + 1]], buf_b, sem_b).wait()
        pltpu.sync_copy(buf_b, acc.at[dst_v.at[j + 1]], add=True)

    plsc.subcore_barrier()
    pltpu.sync_copy(acc.at[pl.ds(sid * rps, rps)],
                    out_hbm.at[cid].at[pl.ds(sid * rps, rps)])


def _deg_body(cpw, rps, dst_hbm, zero_hbm, ones_hbm, out_hbm,
              dst_v, ones_v, acc):
    cid = lax.axis_index("c")
    sid = lax.axis_index("s")
    wid = sid * _NC + cid
    pltpu.sync_copy(dst_hbm.at[pl.ds(wid * cpw, cpw)], dst_v)
    pltpu.sync_copy(ones_hbm, ones_v)
    pltpu.sync_copy(zero_hbm.at[pl.ds(sid * rps, rps)],
                    acc.at[pl.ds(sid * rps, rps)])
    plsc.subcore_barrier()

    @pl.loop(0, cpw)
    def _(j):
        pltpu.sync_copy(ones_v, acc.at[dst_v.at[j]], add=True)

    plsc.subcore_barrier()
    pltpu.sync_copy(acc.at[pl.ds(sid * rps, rps)],
                    out_hbm.at[cid].at[pl.ds(sid * rps, rps)])


def kernel(x, W1, b1, W2, b2, W3, b3, Wg1, bg1, Wg2, bg2, Wg3, bg3,
           Wf, bf, Wd1, bd1, Wd2, bd2, edge_index):
    N, din = x.shape
    L = Wg1.shape[0]
    C = Wd2.shape[1]
    E = edge_index.shape[1]

    RB = 1280
    NP = -(-N // RB) * RB
    if NP - N < _DEGW:
        NP += RB
    GRID = NP // RB
    RPS = NP // _NS

    cpw = -(-E // (_NW * _CH))
    cpw = -(-cpw // _NBUF) * _NBUF
    EP = _NW * cpw * _CH
    NCH = EP // _CH

    src = edge_index[0]
    dst = edge_index[1]
    srcp = jnp.concatenate(
        [src, jnp.zeros((EP - E,), src.dtype)]).reshape(NCH, _CH)
    dstp = jnp.concatenate(
        [dst, jnp.full((EP - E,), N, dst.dtype)]).reshape(NCH, _CH)

    xp = jnp.pad(x, ((0, NP - N), (0, 0)))
    zeros_l = jnp.zeros((NP, L), F32)
    zeros_d = jnp.zeros((NP, _DEGW), F32)
    ones_d = jnp.ones((_CH, _DEGW), F32)

    b1r, b2r, b3r = b1[None, :], b2[None, :], b3[None, :]
    bg1r, bg2r, bg3r = bg1[None, :], bg2[None, :], bg3[None, :]
    bfr, bd1r, bd2r = bf[None, :], bd1[None, :], bd2[None, :]

    init_embed = pl.pallas_call(
        _mlp3_body,
        grid=(GRID,),
        in_specs=[_rows(RB, din), _full(W1.shape), _full((1, 512)),
                  _full(W2.shape), _full((1, 256)),
                  _full(W3.shape), _full((1, L))],
        out_specs=_rows(RB, L),
        out_shape=jax.ShapeDtypeStruct((NP, L), F32),
        compiler_params=_TC_PARAMS,
    )(xp, W1, b1r, W2, b2r, W3, b3r)

    mesh = plsc.VectorSubcoreMesh(core_axis_name="c", subcore_axis_name="s")

    degp = pl.kernel(
        functools.partial(_deg_body, cpw, RPS),
        out_type=jax.ShapeDtypeStruct((_NC, NP, _DEGW), F32),
        mesh=mesh,
        scratch_types=[
            pltpu.VMEM((cpw, _CH), jnp.int32),
            pltpu.VMEM((_CH, _DEGW), F32),
            pltpu.VMEM_SHARED((NP, _DEGW), F32),
        ],
        compiler_params=_SC_PARAMS,
    )(dstp, zeros_d, ones_d)

    dinv16, g = pl.pallas_call(
        _dinv_g1_body,
        grid=(GRID,),
        in_specs=[_rows3(_NC, RB, _DEGW), _rows(RB, L), _full(Wg1.shape)],
        out_specs=[_rows(RB, _DEGW), _rows(RB, L)],
        out_shape=[jax.ShapeDtypeStruct((NP, _DEGW), F32),
                   jax.ShapeDtypeStruct((NP, L), F32)],
        compiler_params=_TC_PARAMS,
    )(degp, init_embed, Wg1)

    edge_call = pl.kernel(
        functools.partial(_edge_body, cpw, RPS),
        out_type=jax.ShapeDtypeStruct((_NC, NP, L), F32),
        mesh=mesh,
        scratch_types=(
            [pltpu.VMEM((cpw, _CH), jnp.int32),
             pltpu.VMEM((cpw, _CH), jnp.int32)]
            + [pltpu.VMEM((_CH, L), F32)] * _NBUF
            + [pltpu.VMEM_SHARED((NP, L), F32),
               pltpu.VMEM_SHARED((NP, L), F32)]
            + [pltpu.SemaphoreType.DMA] * (2 * _NBUF)
        ),
        compiler_params=_SC_PARAMS,
    )

    def post_call(p, g_cur, bgr, wgn):
        return pl.pallas_call(
            _post_body,
            grid=(GRID,),
            in_specs=[_rows3(_NC, RB, L), _rows(RB, L), _rows(RB, _DEGW),
                      _full((1, L)), _full(wgn.shape)],
            out_specs=_rows(RB, L),
            out_shape=jax.ShapeDtypeStruct((NP, L), F32),
            compiler_params=_TC_PARAMS,
        )(p, g_cur, dinv16, bgr, wgn)

    p = edge_call(g, srcp, dstp, zeros_l)
    g = post_call(p, g, bg1r, Wg2)
    p = edge_call(g, srcp, dstp, zeros_l)
    g = post_call(p, g, bg2r, Wg3)
    p = edge_call(g, srcp, dstp, zeros_l)

    out = pl.pallas_call(
        _dec_body,
        grid=(GRID,),
        in_specs=[_rows3(_NC, RB, L), _rows(RB, L), _rows(RB, _DEGW),
                  _full((1, L)), _rows(RB, L), _full(Wf.shape), _full((1, L)),
                  _full(Wd1.shape), _full((1, L)), _full(Wd2.shape),
                  _full((1, C))],
        out_specs=_rows(RB, C),
        out_shape=jax.ShapeDtypeStruct((NP, C), F32),
        compiler_params=_TC_PARAMS,
    )(p, g, dinv16, bg3r, init_embed, Wf, bfr, Wd1, bd1r, Wd2, bd2r)

    return out[:N]


# speedup vs baseline: 1.3289x; 1.0019x over previous
import functools

import jax
import jax.numpy as jnp
from jax import lax
from jax.experimental import pallas as pl
from jax.experimental.pallas import tpu as pltpu
from jax.experimental.pallas import tpu_sc as plsc

F32 = jnp.float32
_HIGH = lax.Precision.DEFAULT

_NC = 2
_NS = 16
_NW = _NC * _NS
_CH = 128
_DEGW = 16

_TC_PARAMS = pltpu.CompilerParams(dimension_semantics=("parallel",))
_SC_PARAMS = pltpu.CompilerParams(use_tc_tiling_on_sc=False)


def _dot(a, b):
    return jnp.dot(a, b, preferred_element_type=F32, precision=_HIGH)


def _mlp3_body(x_ref, w1, b1, w2, b2, w3, b3, o_ref):
    h = jnp.maximum(_dot(x_ref[...], w1[...]) + b1[...], 0.0)
    h = jnp.maximum(_dot(h, w2[...]) + b2[...], 0.0)
    o_ref[...] = _dot(h, w3[...]) + b3[...]


def _dinv_g1_body(degp_ref, emb_ref, wg1, dinv_ref, g1_ref):
    deg = degp_ref[0] + degp_ref[1] + 1.0
    dinv = lax.rsqrt(deg)
    dinv_ref[...] = dinv
    g1_ref[...] = dinv[:, :1] * _dot(emb_ref[...], wg1[...])


def _post_body(p_ref, g_ref, dinv_ref, bg, wgn, gn_ref):
    dinv = dinv_ref[...][:, :1]
    h = jnp.maximum(dinv * (p_ref[0] + p_ref[1] + g_ref[...]) + bg[...], 0.0)
    gn_ref[...] = dinv * _dot(h, wgn[...])


def _dec_body(p_ref, g_ref, dinv_ref, bg, emb_ref, wf, bf, wd1, bd1, wd2, bd2,
              o_ref):
    dinv = dinv_ref[...][:, :1]
    h3 = dinv * (p_ref[0] + p_ref[1] + g_ref[...]) + bg[...]
    cat = jnp.concatenate([emb_ref[...], h3], axis=-1)
    o = _dot(jnp.maximum(cat, 0.0), wf[...]) + bf[...]
    d = jnp.maximum(_dot(o, wd1[...]) + bd1[...], 0.0)
    logits = _dot(d, wd2[...]) + bd2[...]
    m = jnp.max(logits, axis=-1, keepdims=True)
    e = jnp.exp(logits - m)
    o_ref[...] = e / jnp.sum(e, axis=-1, keepdims=True)


def _full(shape):
    return pl.BlockSpec(shape, lambda i: (0,) * len(shape))


def _rows(rb, *rest):
    n = len(rest)
    return pl.BlockSpec((rb,) + rest, lambda i: (i,) + (0,) * n)


def _rows3(lead, rb, *rest):
    n = len(rest)
    return pl.BlockSpec((lead, rb) + rest, lambda i: (0, i) + (0,) * n)


_NBUF = 3


def _edge_body(cpw, rps, g_hbm, src_hbm, dst_hbm, zero_hbm, out_hbm,
               src_v, dst_v, *rest):
    bufs = rest[:_NBUF]
    g_spm, acc = rest[_NBUF], rest[_NBUF + 1]
    gsems = rest[_NBUF + 2:2 * _NBUF + 2]
    ssems = rest[2 * _NBUF + 2:]
    cid = lax.axis_index("c")
    sid = lax.axis_index("s")
    wid = sid * _NC + cid
    pltpu.sync_copy(src_hbm.at[pl.ds(wid * cpw, cpw)], src_v)
    pltpu.sync_copy(dst_hbm.at[pl.ds(wid * cpw, cpw)], dst_v)
    pltpu.sync_copy(g_hbm.at[pl.ds(sid * rps, rps)],
                    g_spm.at[pl.ds(sid * rps, rps)])
    pltpu.sync_copy(zero_hbm.at[pl.ds(sid * rps, rps)],
                    acc.at[pl.ds(sid * rps, rps)])
    plsc.subcore_barrier()

    def gather(j, b):
        return pltpu.make_async_copy(g_spm.at[src_v.at[j]], bufs[b], gsems[b])

    def scatter(j, b):
        return pltpu.make_async_copy(bufs[b], acc.at[dst_v.at[j]], ssems[b])

    def scatter_start(j, b):
        pltpu.async_copy(bufs[b], acc.at[dst_v.at[j]], ssems[b], add=True)

    gather(0, 0).start()
    gather(1, 1).start()
    gather(2, 2).start()
    gather(0, 0).wait()
    scatter_start(0, 0)
    scatter(0, 0).wait()
    gather(3, 0).start()
    gather(1, 1).wait()
    scatter_start(1, 1)
    scatter(1, 1).wait()
    gather(4, 1).start()
    gather(2, 2).wait()
    scatter_start(2, 2)

    @pl.loop(3, cpw - 3, step=3)
    def _(j):
        for t in range(3):
            jj = j + t
            bn = (t + 2) % 3
            scatter(jj - 1, bn).wait()
            gather(jj + 2, bn).start()
            gather(jj, t).wait()
            scatter_start(jj, t)

    scatter(cpw - 4, 2).wait()
    gather(cpw - 1, 2).start()
    for t in range(3):
        gather(cpw - 3 + t, t).wait()
        scatter_start(cpw - 3 + t, t)
    for t in range(3):
        scatter(cpw - 3 + t, t).wait()

    plsc.subcore_barrier()
    pltpu.sync_copy(acc.at[pl.ds(sid * rps, rps)],
                    out_hbm.at[cid].at[pl.ds(sid * rps, rps)])


def _deg_body(cpw, rps, dst_hbm, zero_hbm, ones_hbm, out_hbm,
              dst_v, ones_v, acc):
    cid = lax.axis_index("c")
    sid = lax.axis_index("s")
    wid = sid * _NC + cid
    pltpu.sync_copy(dst_hbm.at[pl.ds(wid * cpw, cpw)], dst_v)
    pltpu.sync_copy(ones_hbm, ones_v)
    pltpu.sync_copy(zero_hbm.at[pl.ds(sid * rps, rps)],
                    acc.at[pl.ds(sid * rps, rps)])
    plsc.subcore_barrier()

    @pl.loop(0, cpw)
    def _(j):
        pltpu.sync_copy(ones_v, acc.at[dst_v.at[j]], add=True)

    plsc.subcore_barrier()
    pltpu.sync_copy(acc.at[pl.ds(sid * rps, rps)],
                    out_hbm.at[cid].at[pl.ds(sid * rps, rps)])


def kernel(x, W1, b1, W2, b2, W3, b3, Wg1, bg1, Wg2, bg2, Wg3, bg3,
           Wf, bf, Wd1, bd1, Wd2, bd2, edge_index):
    N, din = x.shape
    L = Wg1.shape[0]
    C = Wd2.shape[1]
    E = edge_index.shape[1]

    RB = 1280
    NP = -(-N // RB) * RB
    if NP - N < _DEGW:
        NP += RB
    GRID = NP // RB
    RPS = NP // _NS

    cpw = -(-E // (_NW * _CH))
    cpw = -(-cpw // _NBUF) * _NBUF
    EP = _NW * cpw * _CH
    NCH = EP // _CH

    src = edge_index[0]
    dst = edge_index[1]
    srcp = jnp.concatenate(
        [src, jnp.zeros((EP - E,), src.dtype)]).reshape(NCH, _CH)
    dstp = jnp.concatenate(
        [dst, jnp.full((EP - E,), N, dst.dtype)]).reshape(NCH, _CH)

    xp = jnp.pad(x, ((0, NP - N), (0, 0)))
    zeros_l = jnp.zeros((NP, L), F32)
    zeros_d = jnp.zeros((NP, _DEGW), F32)
    ones_d = jnp.ones((_CH, _DEGW), F32)

    b1r, b2r, b3r = b1[None, :], b2[None, :], b3[None, :]
    bg1r, bg2r, bg3r = bg1[None, :], bg2[None, :], bg3[None, :]
    bfr, bd1r, bd2r = bf[None, :], bd1[None, :], bd2[None, :]

    init_embed = pl.pallas_call(
        _mlp3_body,
        grid=(GRID,),
        in_specs=[_rows(RB, din), _full(W1.shape), _full((1, 512)),
                  _full(W2.shape), _full((1, 256)),
                  _full(W3.shape), _full((1, L))],
        out_specs=_rows(RB, L),
        out_shape=jax.ShapeDtypeStruct((NP, L), F32),
        compiler_params=_TC_PARAMS,
    )(xp, W1, b1r, W2, b2r, W3, b3r)

    mesh = plsc.VectorSubcoreMesh(core_axis_name="c", subcore_axis_name="s")

    degp = pl.kernel(
        functools.partial(_deg_body, cpw, RPS),
        out_type=jax.ShapeDtypeStruct((_NC, NP, _DEGW), F32),
        mesh=mesh,
        scratch_types=[
            pltpu.VMEM((cpw, _CH), jnp.int32),
            pltpu.VMEM((_CH, _DEGW), F32),
            pltpu.VMEM_SHARED((NP, _DEGW), F32),
        ],
        compiler_params=_SC_PARAMS,
    )(dstp, zeros_d, ones_d)

    dinv16, g = pl.pallas_call(
        _dinv_g1_body,
        grid=(GRID,),
        in_specs=[_rows3(_NC, RB, _DEGW), _rows(RB, L), _full(Wg1.shape)],
        out_specs=[_rows(RB, _DEGW), _rows(RB, L)],
        out_shape=[jax.ShapeDtypeStruct((NP, _DEGW), F32),
                   jax.ShapeDtypeStruct((NP, L), F32)],
        compiler_params=_TC_PARAMS,
    )(degp, init_embed, Wg1)

    edge_call = pl.kernel(
        functools.partial(_edge_body, cpw, RPS),
        out_type=jax.ShapeDtypeStruct((_NC, NP, L), F32),
        mesh=mesh,
        scratch_types=(
            [pltpu.VMEM((cpw, _CH), jnp.int32),
             pltpu.VMEM((cpw, _CH), jnp.int32)]
            + [pltpu.VMEM((_CH, L), F32)] * _NBUF
            + [pltpu.VMEM_SHARED((NP, L), F32),
               pltpu.VMEM_SHARED((NP, L), F32)]
            + [pltpu.SemaphoreType.DMA] * (2 * _NBUF)
        ),
        compiler_params=_SC_PARAMS,
    )

    def post_call(p, g_cur, bgr, wgn):
        return pl.pallas_call(
            _post_body,
            grid=(GRID,),
            in_specs=[_rows3(_NC, RB, L), _rows(RB, L), _rows(RB, _DEGW),
                      _full((1, L)), _full(wgn.shape)],
            out_specs=_rows(RB, L),
            out_shape=jax.ShapeDtypeStruct((NP, L), F32),
            compiler_params=_TC_PARAMS,
        )(p, g_cur, dinv16, bgr, wgn)

    p = edge_call(g, srcp, dstp, zeros_l)
    g = post_call(p, g, bg1r, Wg2)
    p = edge_call(g, srcp, dstp, zeros_l)
    g = post_call(p, g, bg2r, Wg3)
    p = edge_call(g, srcp, dstp, zeros_l)

    out = pl.pallas_call(
        _dec_body,
        grid=(GRID,),
        in_specs=[_rows3(_NC, RB, L), _rows(RB, L), _rows(RB, _DEGW),
                  _full((1, L)), _rows(RB, L), _full(Wf.shape), _full((1, L)),
                  _full(Wd1.shape), _full((1, L)), _full(Wd2.shape),
                  _full((1, C))],
        out_specs=_rows(RB, C),
        out_shape=jax.ShapeDtypeStruct((NP, C), F32),
        compiler_params=_TC_PARAMS,
    )(p, g, dinv16, bg3r, init_embed, Wf, bfr, Wd1, bd1r, Wd2, bd2r)

    return out[:N]


# speedup vs baseline: 1.3581x; 1.0220x over previous
import functools

import jax
import jax.numpy as jnp
from jax import lax
from jax.experimental import pallas as pl
from jax.experimental.pallas import tpu as pltpu
from jax.experimental.pallas import tpu_sc as plsc

F32 = jnp.float32
_HIGH = lax.Precision.DEFAULT

_NC = 2
_NS = 16
_NW = _NC * _NS
_CH = 128
_DEGW = 16

_TC_PARAMS = pltpu.CompilerParams(dimension_semantics=("parallel",))
_SC_PARAMS = pltpu.CompilerParams(use_tc_tiling_on_sc=False)


def _dot(a, b):
    return jnp.dot(a, b, preferred_element_type=F32, precision=_HIGH)


def _mlp3_body(x_ref, w1, b1, w2, b2, w3, b3, o_ref):
    h = jnp.maximum(_dot(x_ref[...], w1[...]) + b1[...], 0.0)
    h = jnp.maximum(_dot(h, w2[...]) + b2[...], 0.0)
    o_ref[...] = _dot(h, w3[...]) + b3[...]


def _dinv_g1_body(degp_ref, emb_ref, wg1, dinv_ref, g1_ref):
    deg = degp_ref[0] + degp_ref[1] + 1.0
    dinv = lax.rsqrt(deg)
    dinv_ref[...] = dinv
    g1_ref[...] = dinv[:, :1] * _dot(emb_ref[...], wg1[...])


def _post_body(p_ref, g_ref, dinv_ref, bg, wgn, gn_ref):
    dinv = dinv_ref[...][:, :1]
    h = jnp.maximum(dinv * (p_ref[0] + p_ref[1] + g_ref[...]) + bg[...], 0.0)
    gn_ref[...] = dinv * _dot(h, wgn[...])


def _dec_body(p_ref, g_ref, dinv_ref, bg, emb_ref, wf, bf, wd1, bd1, wd2, bd2,
              o_ref):
    dinv = dinv_ref[...][:, :1]
    h3 = dinv * (p_ref[0] + p_ref[1] + g_ref[...]) + bg[...]
    cat = jnp.concatenate([emb_ref[...], h3], axis=-1)
    o = _dot(jnp.maximum(cat, 0.0), wf[...]) + bf[...]
    d = jnp.maximum(_dot(o, wd1[...]) + bd1[...], 0.0)
    logits = _dot(d, wd2[...]) + bd2[...]
    m = jnp.max(logits, axis=-1, keepdims=True)
    e = jnp.exp(logits - m)
    o_ref[...] = e / jnp.sum(e, axis=-1, keepdims=True)


def _full(shape):
    return pl.BlockSpec(shape, lambda i: (0,) * len(shape))


def _rows(rb, *rest):
    n = len(rest)
    return pl.BlockSpec((rb,) + rest, lambda i: (i,) + (0,) * n)


def _rows3(lead, rb, *rest):
    n = len(rest)
    return pl.BlockSpec((lead, rb) + rest, lambda i: (0, i) + (0,) * n)


_NBUF = 3


def _zero_fill(buf, rows, width):
    @pl.loop(0, rows)
    def _(r):
        for c in range(width // 16):
            buf[r, pl.ds(c * 16, 16)] = jnp.zeros((16,), F32)


def _edge_body(cpw, rps, g_hbm, src_hbm, dst_hbm, out_hbm,
               src_v, dst_v, *rest):
    bufs = rest[:_NBUF]
    g_spm, acc = rest[_NBUF], rest[_NBUF + 1]
    gsems = rest[_NBUF + 2:2 * _NBUF + 2]
    ssems = rest[2 * _NBUF + 2:]
    cid = lax.axis_index("c")
    sid = lax.axis_index("s")
    wid = sid * _NC + cid
    _zero_fill(bufs[0], _CH, 64)
    for k in range(rps // _CH):
        pltpu.sync_copy(bufs[0], acc.at[pl.ds(sid * rps + k * _CH, _CH)])
    pltpu.sync_copy(src_hbm.at[wid], src_v)
    pltpu.sync_copy(dst_hbm.at[wid], dst_v)
    pltpu.sync_copy(g_hbm.at[pl.ds(sid * rps, rps)],
                    g_spm.at[pl.ds(sid * rps, rps)])
    plsc.subcore_barrier()

    def gather(j, b):
        return pltpu.make_async_copy(g_spm.at[src_v.at[j]], bufs[b], gsems[b])

    def scatter(j, b):
        return pltpu.make_async_copy(bufs[b], acc.at[dst_v.at[j]], ssems[b])

    def scatter_start(j, b):
        pltpu.async_copy(bufs[b], acc.at[dst_v.at[j]], ssems[b], add=True)

    gather(0, 0).start()
    gather(1, 1).start()
    gather(2, 2).start()
    gather(0, 0).wait()
    scatter_start(0, 0)
    scatter(0, 0).wait()
    gather(3, 0).start()
    gather(1, 1).wait()
    scatter_start(1, 1)
    scatter(1, 1).wait()
    gather(4, 1).start()
    gather(2, 2).wait()
    scatter_start(2, 2)

    @pl.loop(3, cpw - 3, step=3)
    def _(j):
        for t in range(3):
            jj = j + t
            bn = (t + 2) % 3
            scatter(jj - 1, bn).wait()
            gather(jj + 2, bn).start()
            gather(jj, t).wait()
            scatter_start(jj, t)

    scatter(cpw - 4, 2).wait()
    gather(cpw - 1, 2).start()
    for t in range(3):
        gather(cpw - 3 + t, t).wait()
        scatter_start(cpw - 3 + t, t)
    for t in range(3):
        scatter(cpw - 3 + t, t).wait()

    plsc.subcore_barrier()
    pltpu.sync_copy(acc.at[pl.ds(sid * rps, rps)],
                    out_hbm.at[cid].at[pl.ds(sid * rps, rps)])


def _deg_body(cpw, rps, dst_hbm, out_hbm, dst_v, ones_v, zbuf, acc):
    cid = lax.axis_index("c")
    sid = lax.axis_index("s")
    wid = sid * _NC + cid

    @pl.loop(0, _CH)
    def _(r):
        ones_v[r, pl.ds(0, 16)] = jnp.ones((16,), F32)

    _zero_fill(zbuf, _CH, _DEGW)
    for k in range(rps // _CH):
        pltpu.sync_copy(zbuf, acc.at[pl.ds(sid * rps + k * _CH, _CH)])
    pltpu.sync_copy(dst_hbm.at[wid], dst_v)
    plsc.subcore_barrier()

    @pl.loop(0, cpw)
    def _(j):
        pltpu.sync_copy(ones_v, acc.at[dst_v.at[j]], add=True)

    plsc.subcore_barrier()
    pltpu.sync_copy(acc.at[pl.ds(sid * rps, rps)],
                    out_hbm.at[cid].at[pl.ds(sid * rps, rps)])


def kernel(x, W1, b1, W2, b2, W3, b3, Wg1, bg1, Wg2, bg2, Wg3, bg3,
           Wf, bf, Wd1, bd1, Wd2, bd2, edge_index):
    N, din = x.shape
    L = Wg1.shape[0]
    C = Wd2.shape[1]
    E = edge_index.shape[1]

    RB = 1280
    NP = -(-N // RB) * RB
    if NP - N < _DEGW:
        NP += RB
    GRID = NP // RB
    RPS = NP // _NS

    cpw = -(-E // (_NW * _CH))
    cpw = -(-cpw // _NBUF) * _NBUF
    EP = _NW * cpw * _CH
    NCH = EP // _CH

    src = edge_index[0]
    dst = edge_index[1]
    srcp = jnp.concatenate(
        [src, jnp.zeros((EP - E,), src.dtype)]).reshape(_NW, cpw, _CH)
    dstp = jnp.concatenate(
        [dst, jnp.full((EP - E,), N, dst.dtype)]).reshape(_NW, cpw, _CH)

    xp = jnp.pad(x, ((0, NP - N), (0, 0)))

    b1r, b2r, b3r = b1[None, :], b2[None, :], b3[None, :]
    bg1r, bg2r, bg3r = bg1[None, :], bg2[None, :], bg3[None, :]
    bfr, bd1r, bd2r = bf[None, :], bd1[None, :], bd2[None, :]

    init_embed = pl.pallas_call(
        _mlp3_body,
        grid=(GRID,),
        in_specs=[_rows(RB, din), _full(W1.shape), _full((1, 512)),
                  _full(W2.shape), _full((1, 256)),
                  _full(W3.shape), _full((1, L))],
        out_specs=_rows(RB, L),
        out_shape=jax.ShapeDtypeStruct((NP, L), F32),
        compiler_params=_TC_PARAMS,
    )(xp, W1, b1r, W2, b2r, W3, b3r)

    mesh = plsc.VectorSubcoreMesh(core_axis_name="c", subcore_axis_name="s")

    degp = pl.kernel(
        functools.partial(_deg_body, cpw, RPS),
        out_type=jax.ShapeDtypeStruct((_NC, NP, _DEGW), F32),
        mesh=mesh,
        scratch_types=[
            pltpu.VMEM((cpw, _CH), jnp.int32),
            pltpu.VMEM((_CH, _DEGW), F32),
            pltpu.VMEM((_CH, _DEGW), F32),
            pltpu.VMEM_SHARED((NP, _DEGW), F32),
        ],
        compiler_params=_SC_PARAMS,
    )(dstp)

    dinv16, g = pl.pallas_call(
        _dinv_g1_body,
        grid=(GRID,),
        in_specs=[_rows3(_NC, RB, _DEGW), _rows(RB, L), _full(Wg1.shape)],
        out_specs=[_rows(RB, _DEGW), _rows(RB, L)],
        out_shape=[jax.ShapeDtypeStruct((NP, _DEGW), F32),
                   jax.ShapeDtypeStruct((NP, L), F32)],
        compiler_params=_TC_PARAMS,
    )(degp, init_embed, Wg1)

    edge_call = pl.kernel(
        functools.partial(_edge_body, cpw, RPS),
        out_type=jax.ShapeDtypeStruct((_NC, NP, L), F32),
        mesh=mesh,
        scratch_types=(
            [pltpu.VMEM((cpw, _CH), jnp.int32),
             pltpu.VMEM((cpw, _CH), jnp.int32)]
            + [pltpu.VMEM((_CH, L), F32)] * _NBUF
            + [pltpu.VMEM_SHARED((NP, L), F32),
               pltpu.VMEM_SHARED((NP, L), F32)]
            + [pltpu.SemaphoreType.DMA] * (2 * _NBUF)
        ),
        compiler_params=_SC_PARAMS,
    )

    def post_call(p, g_cur, bgr, wgn):
        return pl.pallas_call(
            _post_body,
            grid=(GRID,),
            in_specs=[_rows3(_NC, RB, L), _rows(RB, L), _rows(RB, _DEGW),
                      _full((1, L)), _full(wgn.shape)],
            out_specs=_rows(RB, L),
            out_shape=jax.ShapeDtypeStruct((NP, L), F32),
            compiler_params=_TC_PARAMS,
        )(p, g_cur, dinv16, bgr, wgn)

    p = edge_call(g, srcp, dstp)
    g = post_call(p, g, bg1r, Wg2)
    p = edge_call(g, srcp, dstp)
    g = post_call(p, g, bg2r, Wg3)
    p = edge_call(g, srcp, dstp)

    RBD = 1000
    out = pl.pallas_call(
        _dec_body,
        grid=(N // RBD,),
        in_specs=[_rows3(_NC, RBD, L), _rows(RBD, L), _rows(RBD, _DEGW),
                  _full((1, L)), _rows(RBD, L), _full(Wf.shape), _full((1, L)),
                  _full(Wd1.shape), _full((1, L)), _full(Wd2.shape),
                  _full((1, C))],
        out_specs=_rows(RBD, C),
        out_shape=jax.ShapeDtypeStruct((N, C), F32),
        compiler_params=_TC_PARAMS,
    )(p, g, dinv16, bg3r, init_embed, Wf, bfr, Wd1, bd1r, Wd2, bd2r)

    return out


# speedup vs baseline: 1.4164x; 1.0430x over previous
import functools

import jax
import jax.numpy as jnp
from jax import lax
from jax.experimental import pallas as pl
from jax.experimental.pallas import tpu as pltpu
from jax.experimental.pallas import tpu_sc as plsc

F32 = jnp.float32
_HIGH = lax.Precision.DEFAULT

_NC = 2
_NS = 16
_NW = _NC * _NS
_CH = 128
_DEGW = 16

_TC_PARAMS = pltpu.CompilerParams(dimension_semantics=("parallel",))
_SC_PARAMS = pltpu.CompilerParams(use_tc_tiling_on_sc=False)


def _dot(a, b):
    return jnp.dot(a, b, preferred_element_type=F32, precision=_HIGH)


def _mlp3_body(x_ref, w1, b1, w2, b2, w3, b3, o_ref):
    h = jnp.maximum(_dot(x_ref[...], w1[...]) + b1[...], 0.0)
    h = jnp.maximum(_dot(h, w2[...]) + b2[...], 0.0)
    o_ref[...] = _dot(h, w3[...]) + b3[...]


def _dinv_g1_body(degp_ref, emb_ref, wg1, dinv_ref, g1_ref):
    deg = degp_ref[0] + degp_ref[1] + 1.0
    dinv = lax.rsqrt(deg)
    dinv_ref[...] = dinv
    g1_ref[...] = dinv[:, :1] * _dot(emb_ref[...], wg1[...])


def _post_body(p_ref, g_ref, dinv_ref, bg, wgn, gn_ref):
    dinv = dinv_ref[...][:, :1]
    h = jnp.maximum(dinv * (p_ref[0] + p_ref[1] + g_ref[...]) + bg[...], 0.0)
    gn_ref[...] = dinv * _dot(h, wgn[...])


def _dec_body(p_ref, g_ref, dinv_ref, bg, emb_ref, wf, bf, wd1, bd1, wd2, bd2,
              o_ref):
    dinv = dinv_ref[...][:, :1]
    h3 = dinv * (p_ref[0] + p_ref[1] + g_ref[...]) + bg[...]
    cat = jnp.concatenate([emb_ref[...], h3], axis=-1)
    o = _dot(jnp.maximum(cat, 0.0), wf[...]) + bf[...]
    d = jnp.maximum(_dot(o, wd1[...]) + bd1[...], 0.0)
    logits = _dot(d, wd2[...]) + bd2[...]
    m = jnp.max(logits, axis=-1, keepdims=True)
    e = jnp.exp(logits - m)
    o_ref[...] = e / jnp.sum(e, axis=-1, keepdims=True)


def _full(shape):
    return pl.BlockSpec(shape, lambda i: (0,) * len(shape))


def _rows(rb, *rest):
    n = len(rest)
    return pl.BlockSpec((rb,) + rest, lambda i: (i,) + (0,) * n)


def _rows3(lead, rb, *rest):
    n = len(rest)
    return pl.BlockSpec((lead, rb) + rest, lambda i: (0, i) + (0,) * n)


_NBUF = 3


def _zero_fill(buf, rows, width):
    @pl.loop(0, rows)
    def _(r):
        for c in range(width // 16):
            buf[r, pl.ds(c * 16, 16)] = jnp.zeros((16,), F32)


def _edge_body(cpw, left, rps, g_hbm, ei_hbm, out_hbm,
               src_v, dst_v, *rest):
    bufs = rest[:_NBUF]
    g_spm, acc = rest[_NBUF], rest[_NBUF + 1]
    gsems = rest[_NBUF + 2:2 * _NBUF + 2]
    ssems = rest[2 * _NBUF + 2:]
    cid = lax.axis_index("c")
    sid = lax.axis_index("s")
    wid = sid * _NC + cid
    _zero_fill(bufs[0], _CH, 64)
    for k in range(rps // _CH):
        pltpu.sync_copy(bufs[0], acc.at[pl.ds(sid * rps + k * _CH, _CH)])
    pltpu.sync_copy(ei_hbm.at[0].at[pl.ds(wid * cpw, cpw)],
                    src_v.at[pl.ds(0, cpw)])
    pltpu.sync_copy(ei_hbm.at[1].at[pl.ds(wid * cpw, cpw)],
                    dst_v.at[pl.ds(0, cpw)])
    if left:
        @pl.when(wid < left)
        def _():
            pltpu.sync_copy(ei_hbm.at[0].at[pl.ds(cpw * _NW + wid, 1)],
                            src_v.at[pl.ds(cpw, 1)])
            pltpu.sync_copy(ei_hbm.at[1].at[pl.ds(cpw * _NW + wid, 1)],
                            dst_v.at[pl.ds(cpw, 1)])
    pltpu.sync_copy(g_hbm.at[pl.ds(sid * rps, rps)],
                    g_spm.at[pl.ds(sid * rps, rps)])
    plsc.subcore_barrier()

    def gather(j, b):
        return pltpu.make_async_copy(g_spm.at[src_v.at[j]], bufs[b], gsems[b])

    def scatter(j, b):
        return pltpu.make_async_copy(bufs[b], acc.at[dst_v.at[j]], ssems[b])

    def scatter_start(j, b):
        pltpu.async_copy(bufs[b], acc.at[dst_v.at[j]], ssems[b], add=True)

    cpw3 = cpw - cpw % 3
    gather(0, 0).start()
    gather(1, 1).start()
    gather(2, 2).start()
    gather(0, 0).wait()
    scatter_start(0, 0)
    scatter(0, 0).wait()
    gather(3, 0).start()
    gather(1, 1).wait()
    scatter_start(1, 1)
    scatter(1, 1).wait()
    gather(4, 1).start()
    gather(2, 2).wait()
    scatter_start(2, 2)

    @pl.loop(3, cpw3 - 3, step=3)
    def _(j):
        for t in range(3):
            jj = j + t
            bn = (t + 2) % 3
            scatter(jj - 1, bn).wait()
            gather(jj + 2, bn).start()
            gather(jj, t).wait()
            scatter_start(jj, t)

    scatter(cpw3 - 4, 2).wait()
    gather(cpw3 - 1, 2).start()
    for t in range(3):
        gather(cpw3 - 3 + t, t).wait()
        scatter_start(cpw3 - 3 + t, t)
    for t in range(3):
        scatter(cpw3 - 3 + t, t).wait()

    for j in range(cpw3, cpw):
        pltpu.sync_copy(g_spm.at[src_v.at[j]], bufs[0])
        pltpu.sync_copy(bufs[0], acc.at[dst_v.at[j]], add=True)
    if left:
        @pl.when(wid < left)
        def _():
            pltpu.sync_copy(g_spm.at[src_v.at[cpw]], bufs[0])
            pltpu.sync_copy(bufs[0], acc.at[dst_v.at[cpw]], add=True)

    plsc.subcore_barrier()
    pltpu.sync_copy(acc.at[pl.ds(sid * rps, rps)],
                    out_hbm.at[cid].at[pl.ds(sid * rps, rps)])


def _deg_body(cpw, left, rps, ei_hbm, out_hbm, dst_v, ones_v, zbuf, acc):
    cid = lax.axis_index("c")
    sid = lax.axis_index("s")
    wid = sid * _NC + cid

    @pl.loop(0, _CH)
    def _(r):
        ones_v[r, pl.ds(0, 16)] = jnp.ones((16,), F32)

    _zero_fill(zbuf, _CH, _DEGW)
    for k in range(rps // _CH):
        pltpu.sync_copy(zbuf, acc.at[pl.ds(sid * rps + k * _CH, _CH)])
    pltpu.sync_copy(ei_hbm.at[1].at[pl.ds(wid * cpw, cpw)],
                    dst_v.at[pl.ds(0, cpw)])
    if left:
        @pl.when(wid < left)
        def _():
            pltpu.sync_copy(ei_hbm.at[1].at[pl.ds(cpw * _NW + wid, 1)],
                            dst_v.at[pl.ds(cpw, 1)])
    plsc.subcore_barrier()

    @pl.loop(0, cpw)
    def _(j):
        pltpu.sync_copy(ones_v, acc.at[dst_v.at[j]], add=True)

    if left:
        @pl.when(wid < left)
        def _():
            pltpu.sync_copy(ones_v, acc.at[dst_v.at[cpw]], add=True)

    plsc.subcore_barrier()
    pltpu.sync_copy(acc.at[pl.ds(sid * rps, rps)],
                    out_hbm.at[cid].at[pl.ds(sid * rps, rps)])


def kernel(x, W1, b1, W2, b2, W3, b3, Wg1, bg1, Wg2, bg2, Wg3, bg3,
           Wf, bf, Wd1, bd1, Wd2, bd2, edge_index):
    N, din = x.shape
    L = Wg1.shape[0]
    C = Wd2.shape[1]
    E = edge_index.shape[1]

    RB = 1280
    NP = -(-N // RB) * RB
    if NP - N < _DEGW:
        NP += RB
    GRID = NP // RB
    RPS = NP // _NS

    EP = -(-E // _CH) * _CH
    if EP != E:
        ei = jnp.stack(
            [jnp.concatenate([edge_index[0],
                              jnp.zeros((EP - E,), edge_index.dtype)]),
             jnp.concatenate([edge_index[1],
                              jnp.full((EP - E,), N, edge_index.dtype)])])
    else:
        ei = edge_index
    NCH = EP // _CH
    cpw = NCH // _NW
    left = NCH % _NW
    ei3 = ei.reshape(2, NCH, _CH)

    xp = jnp.pad(x, ((0, NP - N), (0, 0)))

    b1r, b2r, b3r = b1[None, :], b2[None, :], b3[None, :]
    bg1r, bg2r, bg3r = bg1[None, :], bg2[None, :], bg3[None, :]
    bfr, bd1r, bd2r = bf[None, :], bd1[None, :], bd2[None, :]

    init_embed = pl.pallas_call(
        _mlp3_body,
        grid=(GRID,),
        in_specs=[_rows(RB, din), _full(W1.shape), _full((1, 512)),
                  _full(W2.shape), _full((1, 256)),
                  _full(W3.shape), _full((1, L))],
        out_specs=_rows(RB, L),
        out_shape=jax.ShapeDtypeStruct((NP, L), F32),
        compiler_params=_TC_PARAMS,
    )(xp, W1, b1r, W2, b2r, W3, b3r)

    mesh = plsc.VectorSubcoreMesh(core_axis_name="c", subcore_axis_name="s")

    degp = pl.kernel(
        functools.partial(_deg_body, cpw, left, RPS),
        out_type=jax.ShapeDtypeStruct((_NC, NP, _DEGW), F32),
        mesh=mesh,
        scratch_types=[
            pltpu.VMEM((cpw + 1, _CH), jnp.int32),
            pltpu.VMEM((_CH, _DEGW), F32),
            pltpu.VMEM((_CH, _DEGW), F32),
            pltpu.VMEM_SHARED((NP, _DEGW), F32),
        ],
        compiler_params=_SC_PARAMS,
    )(ei3)

    dinv16, g = pl.pallas_call(
        _dinv_g1_body,
        grid=(GRID,),
        in_specs=[_rows3(_NC, RB, _DEGW), _rows(RB, L), _full(Wg1.shape)],
        out_specs=[_rows(RB, _DEGW), _rows(RB, L)],
        out_shape=[jax.ShapeDtypeStruct((NP, _DEGW), F32),
                   jax.ShapeDtypeStruct((NP, L), F32)],
        compiler_params=_TC_PARAMS,
    )(degp, init_embed, Wg1)

    edge_call = pl.kernel(
        functools.partial(_edge_body, cpw, left, RPS),
        out_type=jax.ShapeDtypeStruct((_NC, NP, L), F32),
        mesh=mesh,
        scratch_types=(
            [pltpu.VMEM((cpw + 1, _CH), jnp.int32),
             pltpu.VMEM((cpw + 1, _CH), jnp.int32)]
            + [pltpu.VMEM((_CH, L), F32)] * _NBUF
            + [pltpu.VMEM_SHARED((NP, L), F32),
               pltpu.VMEM_SHARED((NP, L), F32)]
            + [pltpu.SemaphoreType.DMA] * (2 * _NBUF)
        ),
        compiler_params=_SC_PARAMS,
    )

    def post_call(p, g_cur, bgr, wgn):
        return pl.pallas_call(
            _post_body,
            grid=(GRID,),
            in_specs=[_rows3(_NC, RB, L), _rows(RB, L), _rows(RB, _DEGW),
                      _full((1, L)), _full(wgn.shape)],
            out_specs=_rows(RB, L),
            out_shape=jax.ShapeDtypeStruct((NP, L), F32),
            compiler_params=_TC_PARAMS,
        )(p, g_cur, dinv16, bgr, wgn)

    p = edge_call(g, ei3)
    g = post_call(p, g, bg1r, Wg2)
    p = edge_call(g, ei3)
    g = post_call(p, g, bg2r, Wg3)
    p = edge_call(g, ei3)

    RBD = 1000
    out = pl.pallas_call(
        _dec_body,
        grid=(N // RBD,),
        in_specs=[_rows3(_NC, RBD, L), _rows(RBD, L), _rows(RBD, _DEGW),
                  _full((1, L)), _rows(RBD, L), _full(Wf.shape), _full((1, L)),
                  _full(Wd1.shape), _full((1, L)), _full(Wd2.shape),
                  _full((1, C))],
        out_specs=_rows(RBD, C),
        out_shape=jax.ShapeDtypeStruct((N, C), F32),
        compiler_params=_TC_PARAMS,
    )(p, g, dinv16, bg3r, init_embed, Wf, bfr, Wd1, bd1r, Wd2, bd2r)

    return out


# speedup vs baseline: 1.5475x; 1.0925x over previous
import functools

import jax
import jax.numpy as jnp
from jax import lax
from jax.experimental import pallas as pl
from jax.experimental.pallas import tpu as pltpu
from jax.experimental.pallas import tpu_sc as plsc

F32 = jnp.float32
_HIGH = lax.Precision.DEFAULT

_NC = 2
_NS = 16
_NW = _NC * _NS
_CH = 128
_DEGW = 16

_TC_PARAMS = pltpu.CompilerParams(dimension_semantics=("parallel",))
_SC_PARAMS = pltpu.CompilerParams(use_tc_tiling_on_sc=False)


def _dot(a, b):
    return jnp.dot(a, b, preferred_element_type=F32, precision=_HIGH)


def _mlp3_body(x_ref, w1, b1, w2, b2, w3, b3, o_ref):
    h = jnp.maximum(_dot(x_ref[...], w1[...]) + b1[...], 0.0)
    h = jnp.maximum(_dot(h, w2[...]) + b2[...], 0.0)
    o_ref[...] = _dot(h, w3[...]) + b3[...]


def _dinv_g1_body(degp_ref, emb_ref, wg1, dinv_ref, g1_ref):
    degp = degp_ref[...]
    deg = degp[:, :_DEGW] + degp[:, _DEGW:2 * _DEGW] + 1.0
    dinv = lax.rsqrt(deg)
    dinv_ref[...] = dinv
    g1_ref[...] = dinv[:, :1] * _dot(emb_ref[...], wg1[...])


def _psum(p_ref):
    p = p_ref[...]
    return p[:, :64] + p[:, 64:]


def _post_body(p_ref, g_ref, dinv_ref, bg, wgn, gn_ref):
    dinv = dinv_ref[...][:, :1]
    h = jnp.maximum(dinv * (_psum(p_ref) + g_ref[...]) + bg[...], 0.0)
    gn_ref[...] = dinv * _dot(h, wgn[...])


def _dec_body(p_ref, g_ref, dinv_ref, bg, emb_ref, wf, bf, wd1, bd1, wd2, bd2,
              o_ref):
    dinv = dinv_ref[...][:, :1]
    h3 = dinv * (_psum(p_ref) + g_ref[...]) + bg[...]
    cat = jnp.concatenate([emb_ref[...], h3], axis=-1)
    o = _dot(jnp.maximum(cat, 0.0), wf[...]) + bf[...]
    d = jnp.maximum(_dot(o, wd1[...]) + bd1[...], 0.0)
    logits = _dot(d, wd2[...]) + bd2[...]
    m = jnp.max(logits, axis=-1, keepdims=True)
    e = jnp.exp(logits - m)
    o_ref[...] = e / jnp.sum(e, axis=-1, keepdims=True)


def _full(shape):
    return pl.BlockSpec(shape, lambda i: (0,) * len(shape))


def _rows(rb, *rest):
    n = len(rest)
    return pl.BlockSpec((rb,) + rest, lambda i: (i,) + (0,) * n)


def _rows3(lead, rb, *rest):
    n = len(rest)
    return pl.BlockSpec((lead, rb) + rest, lambda i: (0, i) + (0,) * n)


_NBUF = 3


def _zero_fill(buf, rows, width):
    @pl.loop(0, rows)
    def _(r):
        for c in range(width // 16):
            buf[r, pl.ds(c * 16, 16)] = jnp.zeros((16,), F32)


def _edge_body(cpw, left, rps, g_hbm, ei_hbm, out_hbm,
               src_v, dst_v, *rest):
    bufs = rest[:_NBUF]
    g_spm, acc = rest[_NBUF], rest[_NBUF + 1]
    gsems = rest[_NBUF + 2:2 * _NBUF + 2]
    ssems = rest[2 * _NBUF + 2:]
    cid = lax.axis_index("c")
    sid = lax.axis_index("s")
    wid = sid * _NC + cid
    _zero_fill(bufs[0], _CH, 64)
    for k in range(rps // _CH):
        pltpu.sync_copy(bufs[0], acc.at[pl.ds(sid * rps + k * _CH, _CH)])
    pltpu.sync_copy(ei_hbm.at[0].at[pl.ds(wid * cpw, cpw)],
                    src_v.at[pl.ds(0, cpw)])
    pltpu.sync_copy(ei_hbm.at[1].at[pl.ds(wid * cpw, cpw)],
                    dst_v.at[pl.ds(0, cpw)])
    if left:
        @pl.when(wid < left)
        def _():
            pltpu.sync_copy(ei_hbm.at[0].at[pl.ds(cpw * _NW + wid, 1)],
                            src_v.at[pl.ds(cpw, 1)])
            pltpu.sync_copy(ei_hbm.at[1].at[pl.ds(cpw * _NW + wid, 1)],
                            dst_v.at[pl.ds(cpw, 1)])
    pltpu.sync_copy(g_hbm.at[pl.ds(sid * rps, rps)],
                    g_spm.at[pl.ds(sid * rps, rps)])
    plsc.subcore_barrier()

    def gather(j, b):
        return pltpu.make_async_copy(g_spm.at[src_v.at[j]], bufs[b], gsems[b])

    def scatter(j, b):
        return pltpu.make_async_copy(bufs[b], acc.at[dst_v.at[j]], ssems[b])

    def scatter_start(j, b):
        pltpu.async_copy(bufs[b], acc.at[dst_v.at[j]], ssems[b], add=True)

    cpw3 = cpw - cpw % 3
    gather(0, 0).start()
    gather(1, 1).start()
    gather(2, 2).start()
    gather(0, 0).wait()
    scatter_start(0, 0)
    scatter(0, 0).wait()
    gather(3, 0).start()
    gather(1, 1).wait()
    scatter_start(1, 1)
    scatter(1, 1).wait()
    gather(4, 1).start()
    gather(2, 2).wait()
    scatter_start(2, 2)

    @pl.loop(3, cpw3 - 3, step=3)
    def _(j):
        for t in range(3):
            jj = j + t
            bn = (t + 2) % 3
            scatter(jj - 1, bn).wait()
            gather(jj + 2, bn).start()
            gather(jj, t).wait()
            scatter_start(jj, t)

    scatter(cpw3 - 4, 2).wait()
    gather(cpw3 - 1, 2).start()
    for t in range(3):
        gather(cpw3 - 3 + t, t).wait()
        scatter_start(cpw3 - 3 + t, t)
    for t in range(3):
        scatter(cpw3 - 3 + t, t).wait()

    for j in range(cpw3, cpw):
        pltpu.sync_copy(g_spm.at[src_v.at[j]], bufs[0])
        pltpu.sync_copy(bufs[0], acc.at[dst_v.at[j]], add=True)
    if left:
        @pl.when(wid < left)
        def _():
            pltpu.sync_copy(g_spm.at[src_v.at[cpw]], bufs[0])
            pltpu.sync_copy(bufs[0], acc.at[dst_v.at[cpw]], add=True)

    plsc.subcore_barrier()
    pltpu.sync_copy(acc.at[pl.ds(sid * rps, rps)],
                    out_hbm.at[pl.ds(sid * rps, rps), pl.ds(cid * 64, 64)])


def _deg_body(cpw, left, rps, ei_hbm, out_hbm, dst_v, ones_v, zbuf, acc):
    cid = lax.axis_index("c")
    sid = lax.axis_index("s")
    wid = sid * _NC + cid

    @pl.loop(0, _CH)
    def _(r):
        ones_v[r, pl.ds(0, 16)] = jnp.ones((16,), F32)

    _zero_fill(zbuf, _CH, _DEGW)
    for k in range(rps // _CH):
        pltpu.sync_copy(zbuf, acc.at[pl.ds(sid * rps + k * _CH, _CH)])
    pltpu.sync_copy(ei_hbm.at[1].at[pl.ds(wid * cpw, cpw)],
                    dst_v.at[pl.ds(0, cpw)])
    if left:
        @pl.when(wid < left)
        def _():
            pltpu.sync_copy(ei_hbm.at[1].at[pl.ds(cpw * _NW + wid, 1)],
                            dst_v.at[pl.ds(cpw, 1)])
    plsc.subcore_barrier()

    @pl.loop(0, cpw)
    def _(j):
        pltpu.sync_copy(ones_v, acc.at[dst_v.at[j]], add=True)

    if left:
        @pl.when(wid < left)
        def _():
            pltpu.sync_copy(ones_v, acc.at[dst_v.at[cpw]], add=True)

    plsc.subcore_barrier()
    pltpu.sync_copy(acc.at[pl.ds(sid * rps, rps)],
                    out_hbm.at[pl.ds(sid * rps, rps),
                               pl.ds(cid * _DEGW, _DEGW)])


def kernel(x, W1, b1, W2, b2, W3, b3, Wg1, bg1, Wg2, bg2, Wg3, bg3,
           Wf, bf, Wd1, bd1, Wd2, bd2, edge_index):
    N, din = x.shape
    L = Wg1.shape[0]
    C = Wd2.shape[1]
    E = edge_index.shape[1]

    RB = 1280
    NP = -(-N // RB) * RB
    if NP - N < _DEGW:
        NP += RB
    GRID = NP // RB
    RPS = NP // _NS

    EP = -(-E // _CH) * _CH
    if EP != E:
        ei = jnp.stack(
            [jnp.concatenate([edge_index[0],
                              jnp.zeros((EP - E,), edge_index.dtype)]),
             jnp.concatenate([edge_index[1],
                              jnp.full((EP - E,), N, edge_index.dtype)])])
    else:
        ei = edge_index
    NCH = EP // _CH
    cpw = NCH // _NW
    left = NCH % _NW
    ei3 = ei.reshape(2, NCH, _CH)

    xp = jnp.pad(x, ((0, NP - N), (0, 0)))

    b1r, b2r, b3r = b1[None, :], b2[None, :], b3[None, :]
    bg1r, bg2r, bg3r = bg1[None, :], bg2[None, :], bg3[None, :]
    bfr, bd1r, bd2r = bf[None, :], bd1[None, :], bd2[None, :]

    init_embed = pl.pallas_call(
        _mlp3_body,
        grid=(GRID,),
        in_specs=[_rows(RB, din), _full(W1.shape), _full((1, 512)),
                  _full(W2.shape), _full((1, 256)),
                  _full(W3.shape), _full((1, L))],
        out_specs=_rows(RB, L),
        out_shape=jax.ShapeDtypeStruct((NP, L), F32),
        compiler_params=_TC_PARAMS,
    )(xp, W1, b1r, W2, b2r, W3, b3r)

    mesh = plsc.VectorSubcoreMesh(core_axis_name="c", subcore_axis_name="s")

    degp = pl.kernel(
        functools.partial(_deg_body, cpw, left, RPS),
        out_type=jax.ShapeDtypeStruct((NP, 128), F32),
        mesh=mesh,
        scratch_types=[
            pltpu.VMEM((cpw + 1, _CH), jnp.int32),
            pltpu.VMEM((_CH, _DEGW), F32),
            pltpu.VMEM((_CH, _DEGW), F32),
            pltpu.VMEM_SHARED((NP, _DEGW), F32),
        ],
        compiler_params=_SC_PARAMS,
    )(ei3)

    dinv16, g = pl.pallas_call(
        _dinv_g1_body,
        grid=(GRID,),
        in_specs=[_rows(RB, 128), _rows(RB, L), _full(Wg1.shape)],
        out_specs=[_rows(RB, _DEGW), _rows(RB, L)],
        out_shape=[jax.ShapeDtypeStruct((NP, _DEGW), F32),
                   jax.ShapeDtypeStruct((NP, L), F32)],
        compiler_params=_TC_PARAMS,
    )(degp, init_embed, Wg1)

    edge_call = pl.kernel(
        functools.partial(_edge_body, cpw, left, RPS),
        out_type=jax.ShapeDtypeStruct((NP, 2 * L), F32),
        mesh=mesh,
        scratch_types=(
            [pltpu.VMEM((cpw + 1, _CH), jnp.int32),
             pltpu.VMEM((cpw + 1, _CH), jnp.int32)]
            + [pltpu.VMEM((_CH, L), F32)] * _NBUF
            + [pltpu.VMEM_SHARED((NP, L), F32),
               pltpu.VMEM_SHARED((NP, L), F32)]
            + [pltpu.SemaphoreType.DMA] * (2 * _NBUF)
        ),
        compiler_params=_SC_PARAMS,
    )

    def post_call(p, g_cur, bgr, wgn):
        return pl.pallas_call(
            _post_body,
            grid=(GRID,),
            in_specs=[_rows(RB, 2 * L), _rows(RB, L), _rows(RB, _DEGW),
                      _full((1, L)), _full(wgn.shape)],
            out_specs=_rows(RB, L),
            out_shape=jax.ShapeDtypeStruct((NP, L), F32),
            compiler_params=_TC_PARAMS,
        )(p, g_cur, dinv16, bgr, wgn)

    p = edge_call(g, ei3)
    g = post_call(p, g, bg1r, Wg2)
    p = edge_call(g, ei3)
    g = post_call(p, g, bg2r, Wg3)
    p = edge_call(g, ei3)

    RBD = 1000
    out = pl.pallas_call(
        _dec_body,
        grid=(N // RBD,),
        in_specs=[_rows(RBD, 2 * L), _rows(RBD, L), _rows(RBD, _DEGW),
                  _full((1, L)), _rows(RBD, L), _full(Wf.shape), _full((1, L)),
                  _full(Wd1.shape), _full((1, L)), _full(Wd2.shape),
                  _full((1, C))],
        out_specs=_rows(RBD, C),
        out_shape=jax.ShapeDtypeStruct((N, C), F32),
        compiler_params=_TC_PARAMS,
    )(p, g, dinv16, bg3r, init_embed, Wf, bfr, Wd1, bd1r, Wd2, bd2r)

    return out


# speedup vs baseline: 1.5498x; 1.0015x over previous
import functools

import jax
import jax.numpy as jnp
from jax import lax
from jax.experimental import pallas as pl
from jax.experimental.pallas import tpu as pltpu
from jax.experimental.pallas import tpu_sc as plsc

F32 = jnp.float32
_HIGH = lax.Precision.DEFAULT

_NC = 2
_NS = 16
_NW = _NC * _NS
_CH = 128
_DEGW = 16

_TC_PARAMS = pltpu.CompilerParams(dimension_semantics=("parallel",))
_SC_PARAMS = pltpu.CompilerParams(use_tc_tiling_on_sc=False)


def _dot(a, b):
    return jnp.dot(a, b, preferred_element_type=F32, precision=_HIGH)


def _mlp3_body(x_ref, w1, b1, w2, b2, w3, b3, o_ref):
    h = jnp.maximum(_dot(x_ref[...], w1[...]) + b1[...], 0.0)
    h = jnp.maximum(_dot(h, w2[...]) + b2[...], 0.0)
    o_ref[...] = _dot(h, w3[...]) + b3[...]


def _dinv_g1_body(degp_ref, emb_ref, wg1, dinv_ref, g1_ref):
    degp = degp_ref[...]
    deg = degp[:, :_DEGW] + degp[:, _DEGW:2 * _DEGW] + 1.0
    dinv = lax.rsqrt(deg)
    dinv_ref[...] = dinv
    g1_ref[...] = dinv[:, :1] * _dot(emb_ref[...], wg1[...])


def _psum(p_ref):
    p = p_ref[...]
    return p[:, :64] + p[:, 64:]


def _post_body(p_ref, g_ref, dinv_ref, bg, wgn, gn_ref):
    dinv = dinv_ref[...][:, :1]
    h = jnp.maximum(dinv * (_psum(p_ref) + g_ref[...]) + bg[...], 0.0)
    gn_ref[...] = dinv * _dot(h, wgn[...])


def _dec_body(p_ref, g_ref, dinv_ref, bg, emb_ref, wf, bf, wd1, bd1, wd2, bd2,
              o_ref):
    dinv = dinv_ref[...][:, :1]
    h3 = dinv * (_psum(p_ref) + g_ref[...]) + bg[...]
    cat = jnp.concatenate([emb_ref[...], h3], axis=-1)
    o = _dot(jnp.maximum(cat, 0.0), wf[...]) + bf[...]
    d = jnp.maximum(_dot(o, wd1[...]) + bd1[...], 0.0)
    logits = _dot(d, wd2[...]) + bd2[...]
    m = jnp.max(logits, axis=-1, keepdims=True)
    e = jnp.exp(logits - m)
    o_ref[...] = e / jnp.sum(e, axis=-1, keepdims=True)


def _full(shape):
    return pl.BlockSpec(shape, lambda i: (0,) * len(shape))


def _rows(rb, *rest):
    n = len(rest)
    return pl.BlockSpec((rb,) + rest, lambda i: (i,) + (0,) * n)


_NBUF = 3


def _zero_fill(buf, rows, width):
    @pl.loop(0, rows)
    def _(r):
        for c in range(width // 16):
            buf[r, pl.ds(c * 16, 16)] = jnp.zeros((16,), F32)


def _edge_body(cpw, left, rps, g_hbm, ei_hbm, out_hbm,
               src_v, dst_v, *rest):
    bufs = rest[:_NBUF]
    g_spm, acc = rest[_NBUF], rest[_NBUF + 1]
    gsems = rest[_NBUF + 2:2 * _NBUF + 2]
    ssems = rest[2 * _NBUF + 2:]
    cid = lax.axis_index("c")
    sid = lax.axis_index("s")
    wid = sid * _NC + cid
    _zero_fill(bufs[0], _CH, 64)
    for k in range(rps // _CH):
        pltpu.sync_copy(bufs[0], acc.at[pl.ds(sid * rps + k * _CH, _CH)])
    pltpu.sync_copy(ei_hbm.at[0].at[pl.ds(wid * cpw, cpw)],
                    src_v.at[pl.ds(0, cpw)])
    pltpu.sync_copy(ei_hbm.at[1].at[pl.ds(wid * cpw, cpw)],
                    dst_v.at[pl.ds(0, cpw)])
    if left:
        @pl.when(wid < left)
        def _():
            pltpu.sync_copy(ei_hbm.at[0].at[pl.ds(cpw * _NW + wid, 1)],
                            src_v.at[pl.ds(cpw, 1)])
            pltpu.sync_copy(ei_hbm.at[1].at[pl.ds(cpw * _NW + wid, 1)],
                            dst_v.at[pl.ds(cpw, 1)])
    pltpu.sync_copy(g_hbm.at[pl.ds(sid * rps, rps)],
                    g_spm.at[pl.ds(sid * rps, rps)])
    plsc.subcore_barrier()

    def gather(j, b):
        return pltpu.make_async_copy(g_spm.at[src_v.at[j]], bufs[b], gsems[b])

    def scatter(j, b):
        return pltpu.make_async_copy(bufs[b], acc.at[dst_v.at[j]], ssems[b])

    def scatter_start(j, b):
        pltpu.async_copy(bufs[b], acc.at[dst_v.at[j]], ssems[b], add=True)

    cpw3 = cpw - cpw % 3
    gather(0, 0).start()
    gather(1, 1).start()
    gather(2, 2).start()
    gather(0, 0).wait()
    scatter_start(0, 0)
    scatter(0, 0).wait()
    gather(3, 0).start()
    gather(1, 1).wait()
    scatter_start(1, 1)
    scatter(1, 1).wait()
    gather(4, 1).start()
    gather(2, 2).wait()
    scatter_start(2, 2)

    @pl.loop(3, cpw3 - 3, step=3)
    def _(j):
        for t in range(3):
            jj = j + t
            bn = (t + 2) % 3
            scatter(jj - 1, bn).wait()
            gather(jj + 2, bn).start()
            gather(jj, t).wait()
            scatter_start(jj, t)

    scatter(cpw3 - 4, 2).wait()
    gather(cpw3 - 1, 2).start()
    for t in range(3):
        gather(cpw3 - 3 + t, t).wait()
        scatter_start(cpw3 - 3 + t, t)
    for t in range(3):
        scatter(cpw3 - 3 + t, t).wait()

    for j in range(cpw3, cpw):
        pltpu.sync_copy(g_spm.at[src_v.at[j]], bufs[0])
        pltpu.sync_copy(bufs[0], acc.at[dst_v.at[j]], add=True)
    if left:
        @pl.when(wid < left)
        def _():
            pltpu.sync_copy(g_spm.at[src_v.at[cpw]], bufs[0])
            pltpu.sync_copy(bufs[0], acc.at[dst_v.at[cpw]], add=True)

    plsc.subcore_barrier()
    pltpu.sync_copy(acc.at[pl.ds(sid * rps, rps)],
                    out_hbm.at[pl.ds(sid * rps, rps), pl.ds(cid * 64, 64)])


def _deg_body(cpw, left, rps, ei_hbm, out_hbm, dst_v, ones_v, zbuf, acc):
    cid = lax.axis_index("c")
    sid = lax.axis_index("s")
    wid = sid * _NC + cid

    @pl.loop(0, _CH)
    def _(r):
        ones_v[r, pl.ds(0, 16)] = jnp.ones((16,), F32)

    _zero_fill(zbuf, _CH, _DEGW)
    for k in range(rps // _CH):
        pltpu.sync_copy(zbuf, acc.at[pl.ds(sid * rps + k * _CH, _CH)])
    pltpu.sync_copy(ei_hbm.at[1].at[pl.ds(wid * cpw, cpw)],
                    dst_v.at[pl.ds(0, cpw)])
    if left:
        @pl.when(wid < left)
        def _():
            pltpu.sync_copy(ei_hbm.at[1].at[pl.ds(cpw * _NW + wid, 1)],
                            dst_v.at[pl.ds(cpw, 1)])
    plsc.subcore_barrier()

    @pl.loop(0, cpw)
    def _(j):
        pltpu.sync_copy(ones_v, acc.at[dst_v.at[j]], add=True)

    if left:
        @pl.when(wid < left)
        def _():
            pltpu.sync_copy(ones_v, acc.at[dst_v.at[cpw]], add=True)

    plsc.subcore_barrier()
    pltpu.sync_copy(acc.at[pl.ds(sid * rps, rps)],
                    out_hbm.at[pl.ds(sid * rps, rps),
                               pl.ds(cid * _DEGW, _DEGW)])


def kernel(x, W1, b1, W2, b2, W3, b3, Wg1, bg1, Wg2, bg2, Wg3, bg3,
           Wf, bf, Wd1, bd1, Wd2, bd2, edge_index):
    N, din = x.shape
    L = Wg1.shape[0]
    C = Wd2.shape[1]
    E = edge_index.shape[1]

    RB = 1280
    NP = -(-N // RB) * RB
    if NP - N < _DEGW:
        NP += RB
    GRID = NP // RB
    RPS = NP // _NS

    EP = -(-E // _CH) * _CH
    if EP != E:
        ei = jnp.stack(
            [jnp.concatenate([edge_index[0],
                              jnp.zeros((EP - E,), edge_index.dtype)]),
             jnp.concatenate([edge_index[1],
                              jnp.full((EP - E,), N, edge_index.dtype)])])
    else:
        ei = edge_index
    NCH = EP // _CH
    cpw = NCH // _NW
    left = NCH % _NW
    ei3 = ei.reshape(2, NCH, _CH)

    xp = jnp.pad(x, ((0, NP - N), (0, 0)))

    b1r, b2r, b3r = b1[None, :], b2[None, :], b3[None, :]
    bg1r, bg2r, bg3r = bg1[None, :], bg2[None, :], bg3[None, :]
    bfr, bd1r, bd2r = bf[None, :], bd1[None, :], bd2[None, :]

    init_embed = pl.pallas_call(
        _mlp3_body,
        grid=(GRID,),
        in_specs=[_rows(RB, din), _full(W1.shape), _full((1, 512)),
                  _full(W2.shape), _full((1, 256)),
                  _full(W3.shape), _full((1, L))],
        out_specs=_rows(RB, L),
        out_shape=jax.ShapeDtypeStruct((NP, L), F32),
        compiler_params=_TC_PARAMS,
    )(xp, W1, b1r, W2, b2r, W3, b3r)

    mesh = plsc.VectorSubcoreMesh(core_axis_name="c", subcore_axis_name="s")

    degp = pl.kernel(
        functools.partial(_deg_body, cpw, left, RPS),
        out_type=jax.ShapeDtypeStruct((NP, 128), F32),
        mesh=mesh,
        scratch_types=[
            pltpu.VMEM((cpw + 1, _CH), jnp.int32),
            pltpu.VMEM((_CH, _DEGW), F32),
            pltpu.VMEM((_CH, _DEGW), F32),
            pltpu.VMEM_SHARED((NP, _DEGW), F32),
        ],
        compiler_params=_SC_PARAMS,
    )(ei3)

    dinv16, g = pl.pallas_call(
        _dinv_g1_body,
        grid=(GRID,),
        in_specs=[_rows(RB, 128), _rows(RB, L), _full(Wg1.shape)],
        out_specs=[_rows(RB, _DEGW), _rows(RB, L)],
        out_shape=[jax.ShapeDtypeStruct((NP, _DEGW), F32),
                   jax.ShapeDtypeStruct((NP, L), F32)],
        compiler_params=_TC_PARAMS,
    )(degp, init_embed, Wg1)

    edge_call = pl.kernel(
        functools.partial(_edge_body, cpw, left, RPS),
        out_type=jax.ShapeDtypeStruct((NP, 2 * L), F32),
        mesh=mesh,
        scratch_types=(
            [pltpu.VMEM((cpw + 1, _CH), jnp.int32),
             pltpu.VMEM((cpw + 1, _CH), jnp.int32)]
            + [pltpu.VMEM((_CH, L), F32)] * _NBUF
            + [pltpu.VMEM_SHARED((NP, L), F32),
               pltpu.VMEM_SHARED((NP, L), F32)]
            + [pltpu.SemaphoreType.DMA] * (2 * _NBUF)
        ),
        compiler_params=_SC_PARAMS,
    )

    def post_call(p, g_cur, bgr, wgn):
        return pl.pallas_call(
            _post_body,
            grid=(GRID,),
            in_specs=[_rows(RB, 2 * L), _rows(RB, L), _rows(RB, _DEGW),
                      _full((1, L)), _full(wgn.shape)],
            out_specs=_rows(RB, L),
            out_shape=jax.ShapeDtypeStruct((NP, L), F32),
            compiler_params=_TC_PARAMS,
        )(p, g_cur, dinv16, bgr, wgn)

    p = edge_call(g, ei3)
    g = post_call(p, g, bg1r, Wg2)
    p = edge_call(g, ei3)
    g = post_call(p, g, bg2r, Wg3)
    p = edge_call(g, ei3)

    RBD = 1000
    out = pl.pallas_call(
        _dec_body,
        grid=(N // RBD,),
        in_specs=[_rows(RBD, 2 * L), _rows(RBD, L), _rows(RBD, _DEGW),
                  _full((1, L)), _rows(RBD, L), _full(Wf.shape), _full((1, L)),
                  _full(Wd1.shape), _full((1, L)), _full(Wd2.shape),
                  _full((1, C))],
        out_specs=_rows(RBD, C),
        out_shape=jax.ShapeDtypeStruct((N, C), F32),
        compiler_params=_TC_PARAMS,
    )(p, g, dinv16, bg3r, init_embed, Wf, bfr, Wd1, bd1r, Wd2, bd2r)

    return out
